# baseline (speedup 1.0000x reference)
import functools

import jax
import jax.numpy as jnp
from jax import lax
from jax.experimental import pallas as pl
from jax.experimental.pallas import tpu as pltpu
from jax.experimental.pallas import tpu_sc as plsc

NC = 2
NS = 16
NW = NC * NS
CHUNK = 64
RINIT = 128
L = 16


def _mesh():
    return plsc.VectorSubcoreMesh(core_axis_name="c", subcore_axis_name="s")


@functools.lru_cache(maxsize=None)
def _deg_kernel(np_rows, nch, degw):
    @functools.partial(
        pl.kernel,
        mesh=_mesh(),
        out_type=jax.ShapeDtypeStruct((NC, np_rows, degw), jnp.float32),
        scratch_types=[
            pltpu.VMEM((nch, CHUNK), jnp.int32),
            pltpu.VMEM((CHUNK, degw), jnp.float32),
            pltpu.VMEM_SHARED((np_rows, degw), jnp.float32),
        ],
    )
    def kern(dst_hbm, ones_hbm, out, didx, ones_v, acc):
        c = lax.axis_index("c")
        s = lax.axis_index("s")
        wid = c * NS + s
        pltpu.sync_copy(ones_hbm, ones_v)
        pltpu.sync_copy(dst_hbm.at[pl.ds(wid * nch, nch)], didx)

        rpt = np_rows // NS
        r0 = s * rpt
        def init_body(k, _):
            pltpu.sync_copy(ones_v, acc.at[pl.ds(r0 + k * CHUNK, CHUNK)])
            return 0
        lax.fori_loop(0, rpt // CHUNK, init_body, 0)
        plsc.subcore_barrier()

        def body(j, _):
            pltpu.sync_copy(ones_v, acc.at[didx.at[j]], add=True)
            return 0
        lax.fori_loop(0, nch, body, 0)
        plsc.subcore_barrier()

        def wb_body(k, _):
            rb = r0 + k * CHUNK
            pltpu.sync_copy(acc.at[pl.ds(rb, CHUNK)], ones_v)
            pltpu.sync_copy(ones_v, out.at[c, pl.ds(rb, CHUNK)])
            return 0
        lax.fori_loop(0, rpt // CHUNK, wb_body, 0)

    return kern


@functools.lru_cache(maxsize=None)
def _prop_kernel(np_rows, d, nch):
    @functools.partial(
        pl.kernel,
        mesh=_mesh(),
        out_type=jax.ShapeDtypeStruct((NC, np_rows, d), jnp.float32),
        scratch_types=[
            pltpu.VMEM((nch // 2, CHUNK), jnp.int32),
            pltpu.VMEM((nch // 2, CHUNK), jnp.int32),
            pltpu.VMEM((CHUNK, d), jnp.float32),
            pltpu.VMEM((CHUNK, d), jnp.float32),
            pltpu.VMEM_SHARED((np_rows, d), jnp.float32),
            pltpu.SemaphoreType.DMA,
            pltpu.SemaphoreType.DMA,
        ],
    )
    def kern(table, src_hbm, dst_hbm, zeros_hbm, out, sidx, didx, rows0,
             rows1, acc, sem0, sem1):
        c = lax.axis_index("c")
        s = lax.axis_index("s")
        wid = c * NS + s

        rpt = np_rows // NS
        r0 = s * rpt

        @pl.when(c == 0)
        def _():
            def init_body(k, _):
                rb = r0 + k * CHUNK
                pltpu.sync_copy(table.at[pl.ds(rb, CHUNK)], rows0)
                pltpu.sync_copy(rows0, acc.at[pl.ds(rb, CHUNK)])
                return 0
            lax.fori_loop(0, rpt // CHUNK, init_body, 0)

        @pl.when(c == 1)
        def _():
            pltpu.sync_copy(zeros_hbm, rows0)
            def init_body(k, _):
                pltpu.sync_copy(rows0, acc.at[pl.ds(r0 + k * CHUNK, CHUNK)])
                return 0
            lax.fori_loop(0, rpt // CHUNK, init_body, 0)

        plsc.subcore_barrier()

        rows = (rows0, rows1)
        sems = (sem0, sem1)

        def gather(j, b):
            return pltpu.make_async_copy(table.at[sidx.at[j]], rows[b], sems[b])

        def scatter(j, b):
            pltpu.sync_copy(rows[b], acc.at[didx.at[j]], add=True)

        nchp = nch // 2
        nch2 = nchp // 2
        for h in range(2):
            base = wid * nch + h * nchp
            pltpu.sync_copy(src_hbm.at[pl.ds(base, nchp)], sidx)
            pltpu.sync_copy(dst_hbm.at[pl.ds(base, nchp)], didx)
            gather(0, 0).start()

            def body(i, _):
                j0 = 2 * i
                gather(j0 + 1, 1).start()
                gather(j0, 0).wait()
                scatter(j0, 0)

                @pl.when(i < nch2 - 1)
                def _():
                    gather(j0 + 2, 0).start()

                gather(j0 + 1, 1).wait()
                scatter(j0 + 1, 1)
                return 0

            lax.fori_loop(0, nch2, body, 0)
        plsc.subcore_barrier()

        def wb_body(k, _):
            rb = r0 + k * CHUNK
            pltpu.sync_copy(acc.at[pl.ds(rb, CHUNK)], rows0)
            pltpu.sync_copy(rows0, out.at[c, pl.ds(rb, CHUNK)])
            return 0
        lax.fori_loop(0, rpt // CHUNK, wb_body, 0)

    return kern


def _norm_rows(pdeg, x_pad):
    np_rows, d = x_pad.shape
    blk = 1024
    def body(pdeg_ref, x_ref, o_ref, deg_ref):
        deg = jnp.maximum(
            pdeg_ref[0, :, 0:1] + pdeg_ref[1, :, 0:1] - 1.0, 1.0)
        deg_ref[...] = deg
        o_ref[...] = x_ref[...] * lax.rsqrt(deg)
    return pl.pallas_call(
        body,
        grid=(np_rows // blk,),
        in_specs=[
            pl.BlockSpec((NC, blk, 128), lambda i: (0, i, 0)),
            pl.BlockSpec((blk, d), lambda i: (i, 0)),
        ],
        out_specs=[
            pl.BlockSpec((blk, d), lambda i: (i, 0)),
            pl.BlockSpec((blk, 1), lambda i: (i, 0)),
        ],
        out_shape=[
            jax.ShapeDtypeStruct((np_rows, d), jnp.float32),
            jax.ShapeDtypeStruct((np_rows, 1), jnp.float32),
        ],
    )(pdeg, x_pad)


def _dense_layer(p, degc, w, b, *, last):
    _, np_rows, d = p.shape
    blk = 1024
    def body(p_ref, deg_ref, w_ref, b_ref, o_ref):
        deg = deg_ref[...]
        nrm = lax.rsqrt(deg)
        a = p_ref[0] + p_ref[1]
        z = jnp.dot(a, w_ref[...], preferred_element_type=jnp.float32)
        if last:
            o_ref[...] = z * nrm + b_ref[...]
        else:
            o_ref[...] = jnp.maximum(z / deg + nrm * b_ref[...], 0.0)
    return pl.pallas_call(
        body,
        grid=(np_rows // blk,),
        in_specs=[
            pl.BlockSpec((NC, blk, d), lambda i: (0, i, 0)),
            pl.BlockSpec((blk, 1), lambda i: (i, 0)),
            pl.BlockSpec((d, d), lambda i: (0, 0)),
            pl.BlockSpec((1, d), lambda i: (0, 0)),
        ],
        out_specs=pl.BlockSpec((blk, d), lambda i: (i, 0)),
        out_shape=jax.ShapeDtypeStruct((np_rows, d), jnp.float32),
    )(p, degc, w, b.reshape(1, d))


def kernel(x, edge_index, W1, b1, W2, b2, W3, b3):
    n, d = x.shape
    e = edge_index.shape[1]

    np_rows = -(-n // (NS * RINIT)) * NS * RINIT
    nch = -(-e // (NW * CHUNK * 8)) * 8
    ep = NW * CHUNK * nch
    pad = ep - e

    idx = jnp.arange(pad, dtype=jnp.int32)
    src = jnp.concatenate([edge_index[0], idx % n])
    dst = jnp.concatenate([edge_index[1], n + (idx % L)])
    src2 = src.reshape(NW * nch, CHUNK)
    dst2 = dst.reshape(NW * nch, CHUNK)
    x_pad = jnp.pad(x, ((0, np_rows - n), (0, 0)))
    zeros = jnp.zeros((CHUNK, d), jnp.float32)

    ones = jnp.ones((CHUNK, 128), jnp.float32)
    pdeg = _deg_kernel(np_rows, nch, 128)(dst2, ones)

    prop = _prop_kernel(np_rows, d, nch)
    u0, degc = _norm_rows(pdeg, x_pad)
    p1 = prop(u0, src2, dst2, zeros)
    u1 = _dense_layer(p1, degc, W1, b1, last=False)
    p2 = prop(u1, src2, dst2, zeros)
    u2 = _dense_layer(p2, degc, W2, b2, last=False)
    p3 = prop(u2, src2, dst2, zeros)
    return _dense_layer(p3, degc, W3, b3, last=True)[:n]


# speedup vs baseline: 1.0620x; 1.0620x over previous
import functools

import jax
import jax.numpy as jnp
from jax import lax
from jax.experimental import pallas as pl
from jax.experimental.pallas import tpu as pltpu
from jax.experimental.pallas import tpu_sc as plsc

NC = 2
NS = 16
NW = NC * NS
CHUNK = 64
RINIT = 128
L = 16


def _mesh():
    return plsc.VectorSubcoreMesh(core_axis_name="c", subcore_axis_name="s")


@functools.lru_cache(maxsize=None)
def _deg_kernel(np_rows, nch, degw):
    @functools.partial(
        pl.kernel,
        mesh=_mesh(),
        out_type=jax.ShapeDtypeStruct((NC, np_rows, degw), jnp.float32),
        scratch_types=[
            pltpu.VMEM((nch, CHUNK), jnp.int32),
            pltpu.VMEM((CHUNK, degw), jnp.float32),
            pltpu.VMEM_SHARED((np_rows, degw), jnp.float32),
        ],
    )
    def kern(dst_hbm, ones_hbm, out, didx, ones_v, acc):
        c = lax.axis_index("c")
        s = lax.axis_index("s")
        wid = c * NS + s
        pltpu.sync_copy(ones_hbm, ones_v)
        pltpu.sync_copy(dst_hbm.at[pl.ds(wid * nch, nch)], didx)

        rpt = np_rows // NS
        r0 = s * rpt
        def init_body(k, _):
            pltpu.sync_copy(ones_v, acc.at[pl.ds(r0 + k * CHUNK, CHUNK)])
            return 0
        lax.fori_loop(0, rpt // CHUNK, init_body, 0)
        plsc.subcore_barrier()

        def body(j, _):
            pltpu.sync_copy(ones_v, acc.at[didx.at[j]], add=True)
            return 0
        lax.fori_loop(0, nch, body, 0)
        plsc.subcore_barrier()

        def wb_body(k, _):
            rb = r0 + k * CHUNK
            pltpu.sync_copy(acc.at[pl.ds(rb, CHUNK)], ones_v)
            pltpu.sync_copy(ones_v, out.at[c, pl.ds(rb, CHUNK)])
            return 0
        lax.fori_loop(0, rpt // CHUNK, wb_body, 0)

    return kern


@functools.lru_cache(maxsize=None)
def _prop_kernel(np_rows, d, nch):
    @functools.partial(
        pl.kernel,
        mesh=_mesh(),
        out_type=jax.ShapeDtypeStruct((NC, np_rows, d), jnp.float32),
        scratch_types=[
            pltpu.VMEM((nch // 4, CHUNK), jnp.int32),
            pltpu.VMEM((nch // 4, CHUNK), jnp.int32),
            pltpu.VMEM((CHUNK, d), jnp.float32),
            pltpu.VMEM((CHUNK, d), jnp.float32),
            pltpu.VMEM((CHUNK, d), jnp.float32),
            pltpu.VMEM((CHUNK, d), jnp.float32),
            pltpu.VMEM_SHARED((np_rows, d), jnp.float32),
            pltpu.SemaphoreType.DMA,
            pltpu.SemaphoreType.DMA,
            pltpu.SemaphoreType.DMA,
            pltpu.SemaphoreType.DMA,
            pltpu.SemaphoreType.DMA,
            pltpu.SemaphoreType.DMA,
            pltpu.SemaphoreType.DMA,
            pltpu.SemaphoreType.DMA,
        ],
    )
    def kern(table, src_hbm, dst_hbm, zeros_hbm, out, sidx, didx, rows0,
             rows1, rows2, rows3, acc, sg0, sg1, sg2, sg3, ss0, ss1, ss2,
             ss3):
        c = lax.axis_index("c")
        s = lax.axis_index("s")
        wid = c * NS + s

        rpt = np_rows // NS
        r0 = s * rpt

        @pl.when(c == 0)
        def _():
            def init_body(k, _):
                rb = r0 + k * CHUNK
                pltpu.sync_copy(table.at[pl.ds(rb, CHUNK)], rows0)
                pltpu.sync_copy(rows0, acc.at[pl.ds(rb, CHUNK)])
                return 0
            lax.fori_loop(0, rpt // CHUNK, init_body, 0)

        @pl.when(c == 1)
        def _():
            pltpu.sync_copy(zeros_hbm, rows0)
            def init_body(k, _):
                pltpu.sync_copy(rows0, acc.at[pl.ds(r0 + k * CHUNK, CHUNK)])
                return 0
            lax.fori_loop(0, rpt // CHUNK, init_body, 0)

        plsc.subcore_barrier()

        rows = (rows0, rows1, rows2, rows3)
        sgs = (sg0, sg1, sg2, sg3)
        sss = (ss0, ss1, ss2, ss3)
        NB = 4

        def gat(j, b):
            return pltpu.make_async_copy(table.at[sidx.at[j]], rows[b], sgs[b])

        def sca(j, b):
            return pltpu.make_async_copy(rows[b], acc.at[didx.at[j]], sss[b])

        nchp = nch // NB
        for h in range(NB):
            base = wid * nch + h * nchp
            pltpu.sync_copy(src_hbm.at[pl.ds(base, nchp)], sidx)
            pltpu.sync_copy(dst_hbm.at[pl.ds(base, nchp)], didx)
            for k in range(NB):
                gat(k, k).start()

            def body(i, _):
                for k in range(NB):
                    j = NB * i + k
                    gat(j, k).wait()
                    sca(j, k).start(add=True)
                for k in range(NB):
                    j = NB * i + k
                    sca(j, k).wait()

                    @pl.when(j + NB < nchp)
                    def _():
                        gat(j + NB, k).start()
                return 0

            lax.fori_loop(0, nchp // NB, body, 0)
        plsc.subcore_barrier()

        def wb_body(k, _):
            rb = r0 + k * CHUNK
            pltpu.sync_copy(acc.at[pl.ds(rb, CHUNK)], rows0)
            pltpu.sync_copy(rows0, out.at[c, pl.ds(rb, CHUNK)])
            return 0
        lax.fori_loop(0, rpt // CHUNK, wb_body, 0)

    return kern


def _norm_rows(pdeg, x_pad):
    np_rows, d = x_pad.shape
    blk = 1024
    def body(pdeg_ref, x_ref, o_ref, deg_ref):
        deg = jnp.maximum(
            pdeg_ref[0, :, 0:1] + pdeg_ref[1, :, 0:1] - 1.0, 1.0)
        deg_ref[...] = deg
        o_ref[...] = x_ref[...] * lax.rsqrt(deg)
    return pl.pallas_call(
        body,
        grid=(np_rows // blk,),
        in_specs=[
            pl.BlockSpec((NC, blk, 128), lambda i: (0, i, 0)),
            pl.BlockSpec((blk, d), lambda i: (i, 0)),
        ],
        out_specs=[
            pl.BlockSpec((blk, d), lambda i: (i, 0)),
            pl.BlockSpec((blk, 1), lambda i: (i, 0)),
        ],
        out_shape=[
            jax.ShapeDtypeStruct((np_rows, d), jnp.float32),
            jax.ShapeDtypeStruct((np_rows, 1), jnp.float32),
        ],
    )(pdeg, x_pad)


def _dense_layer(p, degc, w, b, *, last):
    _, np_rows, d = p.shape
    blk = 1024
    def body(p_ref, deg_ref, w_ref, b_ref, o_ref):
        deg = deg_ref[...]
        nrm = lax.rsqrt(deg)
        a = p_ref[0] + p_ref[1]
        z = jnp.dot(a, w_ref[...], preferred_element_type=jnp.float32)
        if last:
            o_ref[...] = z * nrm + b_ref[...]
        else:
            o_ref[...] = jnp.maximum(z / deg + nrm * b_ref[...], 0.0)
    return pl.pallas_call(
        body,
        grid=(np_rows // blk,),
        in_specs=[
            pl.BlockSpec((NC, blk, d), lambda i: (0, i, 0)),
            pl.BlockSpec((blk, 1), lambda i: (i, 0)),
            pl.BlockSpec((d, d), lambda i: (0, 0)),
            pl.BlockSpec((1, d), lambda i: (0, 0)),
        ],
        out_specs=pl.BlockSpec((blk, d), lambda i: (i, 0)),
        out_shape=jax.ShapeDtypeStruct((np_rows, d), jnp.float32),
    )(p, degc, w, b.reshape(1, d))


def kernel(x, edge_index, W1, b1, W2, b2, W3, b3):
    n, d = x.shape
    e = edge_index.shape[1]

    np_rows = -(-n // (NS * RINIT)) * NS * RINIT
    nch = -(-e // (NW * CHUNK * 16)) * 16
    ep = NW * CHUNK * nch
    pad = ep - e

    idx = jnp.arange(pad, dtype=jnp.int32)
    src = jnp.concatenate([edge_index[0], idx % n])
    dst = jnp.concatenate([edge_index[1], n + (idx % L)])
    src2 = src.reshape(NW * nch, CHUNK)
    dst2 = dst.reshape(NW * nch, CHUNK)
    x_pad = jnp.pad(x, ((0, np_rows - n), (0, 0)))
    zeros = jnp.zeros((CHUNK, d), jnp.float32)

    ones = jnp.ones((CHUNK, 128), jnp.float32)
    pdeg = _deg_kernel(np_rows, nch, 128)(dst2, ones)

    prop = _prop_kernel(np_rows, d, nch)
    u0, degc = _norm_rows(pdeg, x_pad)
    p1 = prop(u0, src2, dst2, zeros)
    u1 = _dense_layer(p1, degc, W1, b1, last=False)
    p2 = prop(u1, src2, dst2, zeros)
    u2 = _dense_layer(p2, degc, W2, b2, last=False)
    p3 = prop(u2, src2, dst2, zeros)
    return _dense_layer(p3, degc, W3, b3, last=True)[:n]


# speedup vs baseline: 1.0637x; 1.0016x over previous
import functools

import jax
import jax.numpy as jnp
from jax import lax
from jax.experimental import pallas as pl
from jax.experimental.pallas import tpu as pltpu
from jax.experimental.pallas import tpu_sc as plsc

NC = 2
NS = 16
NW = NC * NS
CHUNK = 64
RINIT = 128
L = 16


def _mesh():
    return plsc.VectorSubcoreMesh(core_axis_name="c", subcore_axis_name="s")


@functools.lru_cache(maxsize=None)
def _deg_kernel(np_rows, nch, degw):
    DCH = 128
    NB = 4

    @functools.partial(
        pl.kernel,
        mesh=_mesh(),
        out_type=jax.ShapeDtypeStruct((NC, np_rows, degw), jnp.float32),
        scratch_types=[
            pltpu.VMEM((nch, DCH), jnp.int32),
            pltpu.VMEM((DCH, degw), jnp.float32),
            pltpu.VMEM_SHARED((np_rows, degw), jnp.float32),
            pltpu.SemaphoreType.DMA,
            pltpu.SemaphoreType.DMA,
            pltpu.SemaphoreType.DMA,
            pltpu.SemaphoreType.DMA,
        ],
    )
    def kern(dst_hbm, ones_hbm, out, didx, ones_v, acc, s0, s1, s2, s3):
        c = lax.axis_index("c")
        s = lax.axis_index("s")
        wid = c * NS + s
        pltpu.sync_copy(ones_hbm, ones_v)
        pltpu.sync_copy(dst_hbm.at[pl.ds(wid * nch, nch)], didx)

        rpt = np_rows // NS
        r0 = s * rpt
        def init_body(k, _):
            pltpu.sync_copy(ones_v, acc.at[pl.ds(r0 + k * DCH, DCH)])
            return 0
        lax.fori_loop(0, rpt // DCH, init_body, 0)
        plsc.subcore_barrier()

        sems = (s0, s1, s2, s3)

        def sca(j, k):
            return pltpu.make_async_copy(ones_v, acc.at[didx.at[j]], sems[k])

        for k in range(NB):
            sca(k, k).start(add=True)

        def body(i, _):
            for k in range(NB):
                j = NB * i + k
                sca(j, k).wait()

                @pl.when(j + NB < nch)
                def _():
                    sca(j + NB, k).start(add=True)
            return 0

        lax.fori_loop(0, nch // NB, body, 0)
        plsc.subcore_barrier()

        def wb_body(k, _):
            rb = r0 + k * DCH
            pltpu.sync_copy(acc.at[pl.ds(rb, DCH)], ones_v)
            pltpu.sync_copy(ones_v, out.at[c, pl.ds(rb, DCH)])
            return 0
        lax.fori_loop(0, rpt // DCH, wb_body, 0)

    return kern


@functools.lru_cache(maxsize=None)
def _prop_kernel(np_rows, d, nch):
    @functools.partial(
        pl.kernel,
        mesh=_mesh(),
        out_type=jax.ShapeDtypeStruct((NC, np_rows, d), jnp.float32),
        scratch_types=[
            pltpu.VMEM((nch // 4, CHUNK), jnp.int32),
            pltpu.VMEM((nch // 4, CHUNK), jnp.int32),
            pltpu.VMEM((CHUNK, d), jnp.float32),
            pltpu.VMEM((CHUNK, d), jnp.float32),
            pltpu.VMEM((CHUNK, d), jnp.float32),
            pltpu.VMEM((CHUNK, d), jnp.float32),
            pltpu.VMEM_SHARED((np_rows, d), jnp.float32),
            pltpu.SemaphoreType.DMA,
            pltpu.SemaphoreType.DMA,
            pltpu.SemaphoreType.DMA,
            pltpu.SemaphoreType.DMA,
            pltpu.SemaphoreType.DMA,
            pltpu.SemaphoreType.DMA,
            pltpu.SemaphoreType.DMA,
            pltpu.SemaphoreType.DMA,
        ],
    )
    def kern(table, src_hbm, dst_hbm, zeros_hbm, out, sidx, didx, rows0,
             rows1, rows2, rows3, acc, sg0, sg1, sg2, sg3, ss0, ss1, ss2,
             ss3):
        c = lax.axis_index("c")
        s = lax.axis_index("s")
        wid = c * NS + s

        rpt = np_rows // NS
        r0 = s * rpt

        @pl.when(c == 0)
        def _():
            def init_body(k, _):
                rb = r0 + k * CHUNK
                pltpu.sync_copy(table.at[pl.ds(rb, CHUNK)], rows0)
                pltpu.sync_copy(rows0, acc.at[pl.ds(rb, CHUNK)])
                return 0
            lax.fori_loop(0, rpt // CHUNK, init_body, 0)

        @pl.when(c == 1)
        def _():
            pltpu.sync_copy(zeros_hbm, rows0)
            def init_body(k, _):
                pltpu.sync_copy(rows0, acc.at[pl.ds(r0 + k * CHUNK, CHUNK)])
                return 0
            lax.fori_loop(0, rpt // CHUNK, init_body, 0)

        plsc.subcore_barrier()

        rows = (rows0, rows1, rows2, rows3)
        sgs = (sg0, sg1, sg2, sg3)
        sss = (ss0, ss1, ss2, ss3)
        NB = 4

        def gat(j, b):
            return pltpu.make_async_copy(table.at[sidx.at[j]], rows[b], sgs[b])

        def sca(j, b):
            return pltpu.make_async_copy(rows[b], acc.at[didx.at[j]], sss[b])

        nchp = nch // NB
        for h in range(NB):
            base = wid * nch + h * nchp
            pltpu.sync_copy(src_hbm.at[pl.ds(base, nchp)], sidx)
            pltpu.sync_copy(dst_hbm.at[pl.ds(base, nchp)], didx)
            for k in range(NB):
                gat(k, k).start()

            def body(i, _):
                for k in range(NB):
                    j = NB * i + k
                    gat(j, k).wait()
                    sca(j, k).start(add=True)
                for k in range(NB):
                    j = NB * i + k
                    sca(j, k).wait()

                    @pl.when(j + NB < nchp)
                    def _():
                        gat(j + NB, k).start()
                return 0

            lax.fori_loop(0, nchp // NB, body, 0)
        plsc.subcore_barrier()

        def wb_body(k, _):
            rb = r0 + k * CHUNK
            pltpu.sync_copy(acc.at[pl.ds(rb, CHUNK)], rows0)
            pltpu.sync_copy(rows0, out.at[c, pl.ds(rb, CHUNK)])
            return 0
        lax.fori_loop(0, rpt // CHUNK, wb_body, 0)

    return kern


def _norm_rows(pdeg, x_pad):
    np_rows, d = x_pad.shape
    blk = 1024
    def body(pdeg_ref, x_ref, o_ref, deg_ref):
        deg = jnp.maximum(
            pdeg_ref[0, :, 0:1] + pdeg_ref[1, :, 0:1] - 1.0, 1.0)
        deg_ref[...] = deg
        o_ref[...] = x_ref[...] * lax.rsqrt(deg)
    return pl.pallas_call(
        body,
        grid=(np_rows // blk,),
        in_specs=[
            pl.BlockSpec((NC, blk, 128), lambda i: (0, i, 0)),
            pl.BlockSpec((blk, d), lambda i: (i, 0)),
        ],
        out_specs=[
            pl.BlockSpec((blk, d), lambda i: (i, 0)),
            pl.BlockSpec((blk, 1), lambda i: (i, 0)),
        ],
        out_shape=[
            jax.ShapeDtypeStruct((np_rows, d), jnp.float32),
            jax.ShapeDtypeStruct((np_rows, 1), jnp.float32),
        ],
    )(pdeg, x_pad)


def _dense_layer(p, degc, w, b, *, last):
    _, np_rows, d = p.shape
    blk = 1024
    def body(p_ref, deg_ref, w_ref, b_ref, o_ref):
        deg = deg_ref[...]
        nrm = lax.rsqrt(deg)
        a = p_ref[0] + p_ref[1]
        z = jnp.dot(a, w_ref[...], preferred_element_type=jnp.float32)
        if last:
            o_ref[...] = z * nrm + b_ref[...]
        else:
            o_ref[...] = jnp.maximum(z / deg + nrm * b_ref[...], 0.0)
    return pl.pallas_call(
        body,
        grid=(np_rows // blk,),
        in_specs=[
            pl.BlockSpec((NC, blk, d), lambda i: (0, i, 0)),
            pl.BlockSpec((blk, 1), lambda i: (i, 0)),
            pl.BlockSpec((d, d), lambda i: (0, 0)),
            pl.BlockSpec((1, d), lambda i: (0, 0)),
        ],
        out_specs=pl.BlockSpec((blk, d), lambda i: (i, 0)),
        out_shape=jax.ShapeDtypeStruct((np_rows, d), jnp.float32),
    )(p, degc, w, b.reshape(1, d))


def kernel(x, edge_index, W1, b1, W2, b2, W3, b3):
    n, d = x.shape
    e = edge_index.shape[1]

    np_rows = -(-n // (NS * RINIT)) * NS * RINIT
    nch = -(-e // (NW * CHUNK * 16)) * 16
    ep = NW * CHUNK * nch
    pad = ep - e

    idx = jnp.arange(pad, dtype=jnp.int32)
    src = jnp.concatenate([edge_index[0], idx % n])
    dst = jnp.concatenate([edge_index[1], n + (idx % L)])
    src2 = src.reshape(NW * nch, CHUNK)
    dst2 = dst.reshape(NW * nch, CHUNK)
    x_pad = jnp.pad(x, ((0, np_rows - n), (0, 0)))
    zeros = jnp.zeros((CHUNK, d), jnp.float32)

    dst2d = dst.reshape(NW * (nch // 2), 128)
    ones = jnp.ones((128, 128), jnp.float32)
    pdeg = _deg_kernel(np_rows, nch // 2, 128)(dst2d, ones)

    prop = _prop_kernel(np_rows, d, nch)
    u0, degc = _norm_rows(pdeg, x_pad)
    p1 = prop(u0, src2, dst2, zeros)
    u1 = _dense_layer(p1, degc, W1, b1, last=False)
    p2 = prop(u1, src2, dst2, zeros)
    u2 = _dense_layer(p2, degc, W2, b2, last=False)
    p3 = prop(u2, src2, dst2, zeros)
    return _dense_layer(p3, degc, W3, b3, last=True)[:n]


# speedup vs baseline: 1.0763x; 1.0119x over previous
import functools

import jax
import jax.numpy as jnp
from jax import lax
from jax.experimental import pallas as pl
from jax.experimental.pallas import tpu as pltpu
from jax.experimental.pallas import tpu_sc as plsc

NC = 2
NS = 16
NW = NC * NS
CHUNK = 64
RINIT = 128
L = 16


def _mesh():
    return plsc.VectorSubcoreMesh(core_axis_name="c", subcore_axis_name="s")


@functools.lru_cache(maxsize=None)
def _deg_kernel(np_rows, nch, degw):
    DCH = 128
    NB = 4

    @functools.partial(
        pl.kernel,
        mesh=_mesh(),
        out_type=jax.ShapeDtypeStruct((NC, np_rows, degw), jnp.float32),
        scratch_types=[
            pltpu.VMEM((nch, DCH), jnp.int32),
            pltpu.VMEM((DCH, degw), jnp.float32),
            pltpu.VMEM_SHARED((np_rows, degw), jnp.float32),
            pltpu.SemaphoreType.DMA,
            pltpu.SemaphoreType.DMA,
            pltpu.SemaphoreType.DMA,
            pltpu.SemaphoreType.DMA,
        ],
    )
    def kern(dst_hbm, ones_hbm, out, didx, ones_v, acc, s0, s1, s2, s3):
        c = lax.axis_index("c")
        s = lax.axis_index("s")
        wid = c * NS + s
        pltpu.sync_copy(ones_hbm, ones_v)
        pltpu.sync_copy(dst_hbm.at[pl.ds(wid * nch, nch)], didx)

        rpt = np_rows // NS
        r0 = s * rpt
        def init_body(k, _):
            pltpu.sync_copy(ones_v, acc.at[pl.ds(r0 + k * DCH, DCH)])
            return 0
        lax.fori_loop(0, rpt // DCH, init_body, 0)
        plsc.subcore_barrier()

        sems = (s0, s1, s2, s3)

        def sca(j, k):
            return pltpu.make_async_copy(ones_v, acc.at[didx.at[j]], sems[k])

        for k in range(NB):
            sca(k, k).start(add=True)

        def body(i, _):
            for k in range(NB):
                j = NB * i + k
                sca(j, k).wait()

                @pl.when(j + NB < nch)
                def _():
                    sca(j + NB, k).start(add=True)
            return 0

        lax.fori_loop(0, nch // NB, body, 0)
        plsc.subcore_barrier()

        def wb_body(k, _):
            rb = r0 + k * DCH
            pltpu.sync_copy(acc.at[pl.ds(rb, DCH)], ones_v)
            pltpu.sync_copy(ones_v, out.at[c, pl.ds(rb, DCH)])
            return 0
        lax.fori_loop(0, rpt // DCH, wb_body, 0)

    return kern


@functools.lru_cache(maxsize=None)
def _prop_kernel(np_rows, d, nch):
    @functools.partial(
        pl.kernel,
        mesh=_mesh(),
        out_type=jax.ShapeDtypeStruct((NC, np_rows, d), jnp.float32),
        scratch_types=[
            pltpu.VMEM((nch // 4, CHUNK), jnp.int32),
            pltpu.VMEM((nch // 4, CHUNK), jnp.int32),
            pltpu.VMEM((CHUNK, d), jnp.float32),
            pltpu.VMEM((CHUNK, d), jnp.float32),
            pltpu.VMEM((CHUNK, d), jnp.float32),
            pltpu.VMEM((CHUNK, d), jnp.float32),
            pltpu.VMEM_SHARED((np_rows, d), jnp.float32),
            pltpu.SemaphoreType.DMA,
            pltpu.SemaphoreType.DMA,
            pltpu.SemaphoreType.DMA,
            pltpu.SemaphoreType.DMA,
            pltpu.SemaphoreType.DMA,
            pltpu.SemaphoreType.DMA,
            pltpu.SemaphoreType.DMA,
            pltpu.SemaphoreType.DMA,
        ],
    )
    def kern(table, src_hbm, dst_hbm, zeros_hbm, out, sidx, didx, rows0,
             rows1, rows2, rows3, acc, sg0, sg1, sg2, sg3, ss0, ss1, ss2,
             ss3):
        c = lax.axis_index("c")
        s = lax.axis_index("s")
        wid = c * NS + s

        rpt = np_rows // NS
        r0 = s * rpt

        @pl.when(c == 0)
        def _():
            def init_body(k, _):
                rb = r0 + k * CHUNK
                pltpu.sync_copy(table.at[pl.ds(rb, CHUNK)], rows0)
                pltpu.sync_copy(rows0, acc.at[pl.ds(rb, CHUNK)])
                return 0
            lax.fori_loop(0, rpt // CHUNK, init_body, 0)

        @pl.when(c == 1)
        def _():
            pltpu.sync_copy(zeros_hbm, rows0)
            def init_body(k, _):
                pltpu.sync_copy(rows0, acc.at[pl.ds(r0 + k * CHUNK, CHUNK)])
                return 0
            lax.fori_loop(0, rpt // CHUNK, init_body, 0)

        plsc.subcore_barrier()

        rows = (rows0, rows1, rows2, rows3)
        sgs = (sg0, sg1, sg2, sg3)
        sss = (ss0, ss1, ss2, ss3)
        NB = 4

        def gat(j, b):
            return pltpu.make_async_copy(table.at[sidx.at[j]], rows[b], sgs[b])

        def sca(j, b):
            return pltpu.make_async_copy(rows[b], acc.at[didx.at[j]], sss[b])

        nchp = nch // NB
        for h in range(NB):
            base = wid * nch + h * nchp
            pltpu.sync_copy(src_hbm.at[pl.ds(base, nchp)], sidx)
            pltpu.sync_copy(dst_hbm.at[pl.ds(base, nchp)], didx)
            for k in range(NB):
                gat(k, k).start()

            def body(i, _):
                for k in range(NB):
                    j = NB * i + k
                    gat(j, k).wait()
                    sca(j, k).start(add=True)
                for k in range(NB):
                    j = NB * i + k
                    sca(j, k).wait()

                    @pl.when(j + NB < nchp)
                    def _():
                        gat(j + NB, k).start()
                return 0

            lax.fori_loop(0, nchp // NB, body, 0)
        plsc.subcore_barrier()

        def wb_body(k, _):
            rb = r0 + k * CHUNK
            pltpu.sync_copy(acc.at[pl.ds(rb, CHUNK)], rows0)
            pltpu.sync_copy(rows0, out.at[c, pl.ds(rb, CHUNK)])
            return 0
        lax.fori_loop(0, rpt // CHUNK, wb_body, 0)

    return kern


def _norm_rows(pdeg, x, np_rows):
    _, d = x.shape
    blk = 1024
    def body(pdeg_ref, x_ref, o_ref, deg_ref):
        deg = jnp.maximum(
            pdeg_ref[0, :, 0:1] + pdeg_ref[1, :, 0:1] - 1.0, 1.0)
        deg_ref[...] = deg
        o_ref[...] = x_ref[...] * lax.rsqrt(deg)
    return pl.pallas_call(
        body,
        grid=(np_rows // blk,),
        in_specs=[
            pl.BlockSpec((NC, blk, 128), lambda i: (0, i, 0)),
            pl.BlockSpec((blk, d), lambda i: (i, 0)),
        ],
        out_specs=[
            pl.BlockSpec((blk, d), lambda i: (i, 0)),
            pl.BlockSpec((blk, 1), lambda i: (i, 0)),
        ],
        out_shape=[
            jax.ShapeDtypeStruct((np_rows, d), jnp.float32),
            jax.ShapeDtypeStruct((np_rows, 1), jnp.float32),
        ],
    )(pdeg, x)


def _dense_layer(p, degc, w, b, *, n_out=None):
    _, np_rows, d = p.shape
    last = n_out is not None
    blk = 1024
    def body(p_ref, deg_ref, w_ref, b_ref, o_ref):
        deg = deg_ref[...]
        nrm = lax.rsqrt(deg)
        a = p_ref[0] + p_ref[1]
        z = jnp.dot(a, w_ref[...], preferred_element_type=jnp.float32)
        if last:
            o_ref[...] = z * nrm + b_ref[...]
        else:
            o_ref[...] = jnp.maximum(z / deg + nrm * b_ref[...], 0.0)
    return pl.pallas_call(
        body,
        grid=(np_rows // blk,),
        in_specs=[
            pl.BlockSpec((NC, blk, d), lambda i: (0, i, 0)),
            pl.BlockSpec((blk, 1), lambda i: (i, 0)),
            pl.BlockSpec((d, d), lambda i: (0, 0)),
            pl.BlockSpec((1, d), lambda i: (0, 0)),
        ],
        out_specs=pl.BlockSpec((blk, d), lambda i: (i, 0)),
        out_shape=jax.ShapeDtypeStruct(
            (n_out if last else np_rows, d), jnp.float32),
    )(p, degc, w, b.reshape(1, d))


def kernel(x, edge_index, W1, b1, W2, b2, W3, b3):
    n, d = x.shape
    e = edge_index.shape[1]

    np_rows = -(-n // (NS * RINIT)) * NS * RINIT
    nch = -(-e // (NW * CHUNK * 16)) * 16
    ep = NW * CHUNK * nch
    pad = ep - e

    idx = jnp.arange(pad, dtype=jnp.int32)
    src = jnp.concatenate([edge_index[0], idx % n])
    dst = jnp.concatenate([edge_index[1], n + (idx % L)])
    src2 = src.reshape(NW * nch, CHUNK)
    dst2 = dst.reshape(NW * nch, CHUNK)
    zeros = jnp.zeros((CHUNK, d), jnp.float32)

    dst2d = dst.reshape(NW * (nch // 2), 128)
    ones = jnp.ones((128, 128), jnp.float32)
    pdeg = _deg_kernel(np_rows, nch // 2, 128)(dst2d, ones)

    prop = _prop_kernel(np_rows, d, nch)
    u0, degc = _norm_rows(pdeg, x, np_rows)
    p1 = prop(u0, src2, dst2, zeros)
    u1 = _dense_layer(p1, degc, W1, b1)
    p2 = prop(u1, src2, dst2, zeros)
    u2 = _dense_layer(p2, degc, W2, b2)
    p3 = prop(u2, src2, dst2, zeros)
    return _dense_layer(p3, degc, W3, b3, n_out=n)


# speedup vs baseline: 1.1192x; 1.0398x over previous
import functools

import jax
import jax.numpy as jnp
from jax import lax
from jax.experimental import pallas as pl
from jax.experimental.pallas import tpu as pltpu
from jax.experimental.pallas import tpu_sc as plsc

NC = 2
NS = 16
NW = NC * NS
CHUNK = 64
RINIT = 128
L = 16


def _mesh():
    return plsc.VectorSubcoreMesh(core_axis_name="c", subcore_axis_name="s")


@functools.lru_cache(maxsize=None)
def _deg_kernel(np_rows, nch, degw):
    DCH = 128
    NB = 4

    @functools.partial(
        pl.kernel,
        mesh=_mesh(),
        out_type=jax.ShapeDtypeStruct((NC, np_rows, degw), jnp.float32),
        scratch_types=[
            pltpu.VMEM((nch, DCH), jnp.int32),
            pltpu.VMEM((DCH, degw), jnp.float32),
            pltpu.VMEM_SHARED((np_rows, degw), jnp.float32),
            pltpu.SemaphoreType.DMA,
            pltpu.SemaphoreType.DMA,
            pltpu.SemaphoreType.DMA,
            pltpu.SemaphoreType.DMA,
        ],
    )
    def kern(dst_hbm, ones_hbm, out, didx, ones_v, acc, s0, s1, s2, s3):
        c = lax.axis_index("c")
        s = lax.axis_index("s")
        wid = c * NS + s
        pltpu.sync_copy(ones_hbm, ones_v)
        pltpu.sync_copy(dst_hbm.at[pl.ds(wid * nch, nch)], didx)

        rpt = np_rows // NS
        r0 = s * rpt
        def init_body(k, _):
            pltpu.sync_copy(ones_v, acc.at[pl.ds(r0 + k * DCH, DCH)])
            return 0
        lax.fori_loop(0, rpt // DCH, init_body, 0)
        plsc.subcore_barrier()

        sems = (s0, s1, s2, s3)

        def sca(j, k):
            return pltpu.make_async_copy(ones_v, acc.at[didx.at[j]], sems[k])

        for k in range(NB):
            sca(k, k).start(add=True)

        def body(i, _):
            for k in range(NB):
                j = NB * i + k
                sca(j, k).wait()

                @pl.when(j + NB < nch)
                def _():
                    sca(j + NB, k).start(add=True)
            return 0

        lax.fori_loop(0, nch // NB, body, 0)
        plsc.subcore_barrier()

        def wb_body(k, _):
            rb = r0 + k * DCH
            pltpu.sync_copy(acc.at[pl.ds(rb, DCH)], ones_v)
            pltpu.sync_copy(ones_v, out.at[c, pl.ds(rb, DCH)])
            return 0
        lax.fori_loop(0, rpt // DCH, wb_body, 0)

    return kern


@functools.lru_cache(maxsize=None)
def _prop_kernel(np_rows, d, nch):
    @functools.partial(
        pl.kernel,
        mesh=_mesh(),
        out_type=jax.ShapeDtypeStruct((NC, np_rows, d), jnp.float32),
        scratch_types=[
            pltpu.VMEM((nch // 4, CHUNK), jnp.int32),
            pltpu.VMEM((nch // 4, CHUNK), jnp.int32),
            pltpu.VMEM((CHUNK, d), jnp.float32),
            pltpu.VMEM((CHUNK, d), jnp.float32),
            pltpu.VMEM((CHUNK, d), jnp.float32),
            pltpu.VMEM((CHUNK, d), jnp.float32),
            pltpu.VMEM_SHARED((np_rows, d), jnp.float32),
            pltpu.SemaphoreType.DMA,
            pltpu.SemaphoreType.DMA,
            pltpu.SemaphoreType.DMA,
            pltpu.SemaphoreType.DMA,
            pltpu.SemaphoreType.DMA,
            pltpu.SemaphoreType.DMA,
            pltpu.SemaphoreType.DMA,
            pltpu.SemaphoreType.DMA,
        ],
    )
    def kern(table, src_hbm, dst_hbm, zeros_hbm, out, sidx, didx, rows0,
             rows1, rows2, rows3, acc, sg0, sg1, sg2, sg3, ss0, ss1, ss2,
             ss3):
        c = lax.axis_index("c")
        s = lax.axis_index("s")
        wid = c * NS + s

        rpt = np_rows // NS
        r0 = s * rpt
        pltpu.sync_copy(zeros_hbm, rows0)
        def init_body(k, _):
            pltpu.sync_copy(rows0, acc.at[pl.ds(r0 + k * CHUNK, CHUNK)])
            return 0
        lax.fori_loop(0, rpt // CHUNK, init_body, 0)
        plsc.subcore_barrier()

        rows = (rows0, rows1, rows2, rows3)
        sgs = (sg0, sg1, sg2, sg3)
        sss = (ss0, ss1, ss2, ss3)
        NB = 4

        def gat(j, b):
            return pltpu.make_async_copy(table.at[sidx.at[j]], rows[b], sgs[b])

        def sca(j, b):
            return pltpu.make_async_copy(rows[b], acc.at[didx.at[j]], sss[b])

        nchp = nch // NB
        for h in range(NB):
            base = wid * nch + h * nchp
            pltpu.sync_copy(src_hbm.at[pl.ds(base, nchp)], sidx)
            pltpu.sync_copy(dst_hbm.at[pl.ds(base, nchp)], didx)
            for k in range(NB):
                gat(k, k).start()

            def body(i, _):
                for k in range(NB):
                    j = NB * i + k
                    gat(j, k).wait()
                    sca(j, k).start(add=True)
                for k in range(NB):
                    j = NB * i + k
                    sca(j, k).wait()

                    @pl.when(j + NB < nchp)
                    def _():
                        gat(j + NB, k).start()
                return 0

            lax.fori_loop(0, nchp // NB, body, 0)
        plsc.subcore_barrier()

        def wb_body(k, _):
            rb = r0 + k * CHUNK
            pltpu.sync_copy(acc.at[pl.ds(rb, CHUNK)], rows0)
            pltpu.sync_copy(rows0, out.at[c, pl.ds(rb, CHUNK)])
            return 0
        lax.fori_loop(0, rpt // CHUNK, wb_body, 0)

    return kern


def _norm_rows(pdeg, x, np_rows):
    _, d = x.shape
    blk = 1024
    def body(pdeg_ref, x_ref, o_ref, deg_ref):
        deg = jnp.maximum(
            pdeg_ref[0, :, 0:1] + pdeg_ref[1, :, 0:1] - 1.0, 1.0)
        deg_ref[...] = deg
        o_ref[...] = x_ref[...] * lax.rsqrt(deg)
    return pl.pallas_call(
        body,
        grid=(np_rows // blk,),
        in_specs=[
            pl.BlockSpec((NC, blk, 128), lambda i: (0, i, 0)),
            pl.BlockSpec((blk, d), lambda i: (i, 0)),
        ],
        out_specs=[
            pl.BlockSpec((blk, d), lambda i: (i, 0)),
            pl.BlockSpec((blk, 1), lambda i: (i, 0)),
        ],
        out_shape=[
            jax.ShapeDtypeStruct((np_rows, d), jnp.float32),
            jax.ShapeDtypeStruct((np_rows, 1), jnp.float32),
        ],
    )(pdeg, x)


def _dense_layer(p, u, degc, w, b, *, n_out=None):
    _, np_rows, d = p.shape
    last = n_out is not None
    blk = 1024
    def body(p_ref, u_ref, deg_ref, w_ref, b_ref, o_ref):
        deg = deg_ref[...]
        nrm = lax.rsqrt(deg)
        a = p_ref[0] + p_ref[1] + u_ref[...]
        z = jnp.dot(a, w_ref[...], preferred_element_type=jnp.float32)
        if last:
            o_ref[...] = z * nrm + b_ref[...]
        else:
            o_ref[...] = jnp.maximum(z / deg + nrm * b_ref[...], 0.0)
    return pl.pallas_call(
        body,
        grid=(np_rows // blk,),
        in_specs=[
            pl.BlockSpec((NC, blk, d), lambda i: (0, i, 0)),
            pl.BlockSpec((blk, d), lambda i: (i, 0)),
            pl.BlockSpec((blk, 1), lambda i: (i, 0)),
            pl.BlockSpec((d, d), lambda i: (0, 0)),
            pl.BlockSpec((1, d), lambda i: (0, 0)),
        ],
        out_specs=pl.BlockSpec((blk, d), lambda i: (i, 0)),
        out_shape=jax.ShapeDtypeStruct(
            (n_out if last else np_rows, d), jnp.float32),
    )(p, u, degc, w, b.reshape(1, d))


def kernel(x, edge_index, W1, b1, W2, b2, W3, b3):
    n, d = x.shape
    e = edge_index.shape[1]

    np_rows = -(-n // (NS * RINIT)) * NS * RINIT
    nch = -(-e // (NW * CHUNK * 16)) * 16
    ep = NW * CHUNK * nch
    pad = ep - e

    idx = jnp.arange(pad, dtype=jnp.int32)
    src = jnp.concatenate([edge_index[0], idx % n])
    dst = jnp.concatenate([edge_index[1], n + (idx % L)])
    src2 = src.reshape(NW * nch, CHUNK)
    dst2 = dst.reshape(NW * nch, CHUNK)
    zeros = jnp.zeros((CHUNK, d), jnp.float32)

    dst2d = dst.reshape(NW * (nch // 2), 128)
    ones = jnp.ones((128, 128), jnp.float32)
    pdeg = _deg_kernel(np_rows, nch // 2, 128)(dst2d, ones)

    prop = _prop_kernel(np_rows, d, nch)
    u0, degc = _norm_rows(pdeg, x, np_rows)
    p1 = prop(u0, src2, dst2, zeros)
    u1 = _dense_layer(p1, u0, degc, W1, b1)
    p2 = prop(u1, src2, dst2, zeros)
    u2 = _dense_layer(p2, u1, degc, W2, b2)
    p3 = prop(u2, src2, dst2, zeros)
    return _dense_layer(p3, u2, degc, W3, b3, n_out=n)
